# Initial kernel scaffold; baseline (speedup 1.0000x reference)
import jax
import jax.numpy as jnp
from jax.experimental import pallas as pl


def kernel(x, edge_index, Wl0, bl0, Wr0, br0, att0, cb0, g0, be0, Wl1, bl1, Wr1, br1, att1, cb1, g1, be1, Wl2, bl2, Wr2, br2, att2, cb2, g2, be2, Wc1, bc1, Wc2, bc2):
    raise NotImplementedError("write your pallas kernel here")


# speedup vs baseline: 5.9920x; 5.9920x over previous
import jax
import jax.numpy as jnp
from jax import lax
from jax.experimental import pallas as pl
from jax.experimental.pallas import tpu as pltpu
from jax.experimental.pallas import tpu_sc as plsc

N_NODES = 10000
NP = 10240
DUMMY = 10000
E_RAW = 320000
E2 = E_RAW + N_NODES
E2P = 331776
NC, NS = 2, 16
NW = NC * NS
CPW = E2P // NW
K1 = 64
K2 = 128
D = 64
PW = 128
HNP = NP // 2
AROWS = 5632
TRASH = HNP


def _mm_body(x_ref, w_ref, b_ref, o_ref):
    o_ref[0] = jnp.dot(x_ref[...], w_ref[0],
                       preferred_element_type=jnp.float32) + b_ref[0]


def _pair_matmul(x, w, b, heads):
    indim = x.shape[1]
    blk = 512
    if heads == 1:
        w = jnp.pad(w, ((0, 0), (0, D)))
        b = jnp.pad(b, (0, D))
    npair = max(heads // 2, 1)
    w3 = w.reshape(indim, npair, PW).transpose(1, 0, 2)
    b3 = b.reshape(1, npair, PW).transpose(1, 0, 2)
    return pl.pallas_call(
        _mm_body,
        grid=(npair, NP // blk),
        in_specs=[
            pl.BlockSpec((blk, indim), lambda p, i: (i, 0)),
            pl.BlockSpec((1, indim, PW), lambda p, i: (p, 0, 0)),
            pl.BlockSpec((1, 1, PW), lambda p, i: (p, 0, 0)),
        ],
        out_specs=pl.BlockSpec((1, blk, PW), lambda p, i: (p, i, 0)),
        out_shape=jax.ShapeDtypeStruct((npair, NP, PW), jnp.float32),
    )(x, w3, b3)


def _colmax_body(a_ref, o_ref):
    @pl.when(pl.program_id(0) == 0)
    def _():
        o_ref[...] = jnp.full_like(o_ref, -jnp.inf)
    blk = a_ref.shape[0]
    m = jnp.max(a_ref[...].reshape(blk // 8, 8, 16), axis=0)
    o_ref[...] = jnp.maximum(o_ref[...], m)


def _colmax(alpha):
    blk = 4096
    return pl.pallas_call(
        _colmax_body,
        grid=(E2P // blk,),
        in_specs=[pl.BlockSpec((blk, 16), lambda i: (i, 0))],
        out_specs=pl.BlockSpec((8, 16), lambda i: (0, 0)),
        out_shape=jax.ShapeDtypeStruct((8, 16), jnp.float32),
    )(alpha)


def _make_pass1(heads, npair):
    def body(*refs):
        xl = refs[0:npair]
        xr = refs[npair:2 * npair]
        att_hbm, src_hbm, dst_hbm = refs[2 * npair:2 * npair + 3]
        alpha_hbm = refs[2 * npair + 3]
        srcv, dstv, gxl, gxr, attv, abuf, gsem = refs[2 * npair + 4:]
        c = lax.axis_index("c")
        s = lax.axis_index("s")
        base0 = (c * NS + s) * CPW
        pltpu.sync_copy(att_hbm, attv)
        attvals = [[attv[h, pl.ds(q * 16, 16)] for q in range(4)]
                   for h in range(heads)]
        lanes = lax.iota(jnp.int32, 16)
        shuf = [lanes ^ bit for bit in (8, 4, 2, 1)]
        dnums = lax.GatherDimensionNumbers(
            offset_dims=(), collapsed_slice_dims=(0,), start_index_map=(0,))

        def hsum(v):
            for ix in shuf:
                v = v + lax.gather(
                    v, ix[:, None], dnums, (1,),
                    mode=lax.GatherScatterMode.PROMISE_IN_BOUNDS)
            return v

        def chunk(k, carry):
            base = base0 + k * K1
            pltpu.sync_copy(src_hbm.at[pl.ds(base, K1)], srcv)
            pltpu.sync_copy(dst_hbm.at[pl.ds(base, K1)], dstv)
            cps = []
            for p in range(npair):
                cps.append(pltpu.async_copy(xl[p].at[srcv], gxl.at[p], gsem))
                cps.append(pltpu.async_copy(xr[p].at[dstv], gxr.at[p], gsem))
            for cp in cps:
                cp.wait()

            def edge(e, ecarry):
                av = jnp.zeros((16,), jnp.float32)
                for h in range(heads):
                    p, half = divmod(h, 2)
                    off = half * D
                    acc = jnp.zeros((16,), jnp.float32)
                    for q in range(4):
                        sl = pl.ds(off + q * 16, 16)
                        sv = gxl[p, e, sl] + gxr[p, e, sl]
                        lr = jnp.maximum(sv, 0.2 * sv)
                        acc = acc + lr * attvals[h][q]
                    av = jnp.where(lanes == h, hsum(acc), av)
                abuf[e] = av
                return ecarry

            lax.fori_loop(0, K1, edge, 0)
            pltpu.sync_copy(abuf, alpha_hbm.at[pl.ds(base, K1)])
            return carry

        lax.fori_loop(0, CPW // K1, chunk, 0)
    return body


def _pass1(xls, xrs, att, src, dst, heads):
    npair = len(xls)
    fn = pl.kernel(
        _make_pass1(heads, npair),
        out_type=jax.ShapeDtypeStruct((E2P, 16), jnp.float32),
        mesh=plsc.VectorSubcoreMesh(core_axis_name="c", subcore_axis_name="s"),
        scratch_types=[
            pltpu.VMEM((K1,), jnp.int32),
            pltpu.VMEM((K1,), jnp.int32),
            pltpu.VMEM((npair, K1, PW), jnp.float32),
            pltpu.VMEM((npair, K1, PW), jnp.float32),
            pltpu.VMEM((heads, D), jnp.float32),
            pltpu.VMEM((K1, 16), jnp.float32),
            pltpu.SemaphoreType.DMA,
        ],
    )
    return fn(*xls, *xrs, att, src, dst)


def _make_pass2(npair):
    def body(*refs):
        xl = refs[0:npair]
        src_hbm, dst_hbm, alpha_hbm, gmax_hbm, outp_hbm = \
            refs[npair:npair + 5]
        srcv, dstv, dstw, gbuf, sbuf, zbuf, exv, gmv, fbuf, acc, gsem = \
            refs[npair + 5:]
        c = lax.axis_index("c")
        s = lax.axis_index("s")
        base0 = (c * NS + s) * CPW
        pltpu.sync_copy(gmax_hbm, gmv)
        gmvec = gmv[...]
        zeros16 = jnp.zeros((16,), jnp.float32)

        def zrow(i, carry):
            for qq in range(PW // 16):
                zbuf[i, pl.ds(qq * 16, 16)] = zeros16
                sbuf[i, pl.ds(qq * 16, 16)] = zeros16
            return carry

        lax.fori_loop(0, K2, zrow, 0)

        def zero_acc():
            def zcp(j, carry):
                pltpu.sync_copy(zbuf.at[pl.ds(0, 88)],
                                acc.at[pl.ds(s * 352 + j * 88, 88)])
                return carry
            lax.fori_loop(0, 352 // 88, zcp, 0)
            plsc.subcore_barrier()

        def flush_acc(slot, r):
            plsc.subcore_barrier()

            def fcp(j, carry):
                r0 = s * 320 + j * 80
                pltpu.sync_copy(acc.at[pl.ds(r0, 80)], fbuf.at[pl.ds(0, 80)])
                pltpu.sync_copy(fbuf.at[pl.ds(0, 80)],
                                outp_hbm.at[c, slot,
                                            pl.ds(r * HNP + r0, 80)])
                return carry

            lax.fori_loop(0, 320 // 80, fcp, 0)
            plsc.subcore_barrier()

        def load_remap_dst(base, lo):
            pltpu.sync_copy(dst_hbm.at[pl.ds(base, K2)], dstv)

            def remap(g, carry):
                d = dstv[pl.ds(g * 16, 16)] - lo
                oob = (d < 0) | (d >= HNP)
                dstw[pl.ds(g * 16, 16)] = jnp.where(oob, TRASH, d)
                return carry

            lax.fori_loop(0, K2 // 16, remap, 0)

        for r in range(2):
            lo = r * HNP

            zero_acc()

            def dchunk(k, carry):
                base = base0 + k * K2
                load_remap_dst(base, lo)
                pltpu.sync_copy(alpha_hbm.at[pl.ds(base, K2)], exv)

                def vexp(e, gc):
                    sbuf[e, pl.ds(0, 16)] = jnp.exp(exv[e] - gmvec)
                    return gc

                lax.fori_loop(0, K2, vexp, 0)
                pltpu.sync_copy(sbuf, acc.at[dstw], add=True)
                return carry

            lax.fori_loop(0, CPW // K2, dchunk, 0)
            flush_acc(npair, r)

            def zden(i, carry):
                sbuf[i, pl.ds(0, 16)] = zeros16
                return carry

            lax.fori_loop(0, K2, zden, 0)

            for p in range(npair):
                zero_acc()

                def chunk(k, carry):
                    base = base0 + k * K2
                    pltpu.sync_copy(src_hbm.at[pl.ds(base, K2)], srcv)
                    load_remap_dst(base, lo)
                    pltpu.sync_copy(alpha_hbm.at[pl.ds(base, K2)], exv)
                    pltpu.async_copy(xl[p].at[srcv], gbuf, gsem).wait()

                    def vexp(e, gc):
                        exv[e] = jnp.exp(exv[e] - gmvec)
                        return gc

                    lax.fori_loop(0, K2, vexp, 0)

                    def edge(e, ec):
                        ev = exv[e]
                        ex0 = ev[2 * p]
                        ex1 = ev[2 * p + 1]
                        for q in range(8):
                            exq = ex0 if q < 4 else ex1
                            sbuf[e, pl.ds(q * 16, 16)] = \
                                gbuf[e, pl.ds(q * 16, 16)] * exq
                        return ec

                    lax.fori_loop(0, K2, edge, 0)
                    pltpu.sync_copy(sbuf, acc.at[dstw], add=True)
                    return carry

                lax.fori_loop(0, CPW // K2, chunk, 0)
                flush_acc(p, r)
    return body


def _pass2(xls, src, dst, alpha, gmax16):
    npair = len(xls)
    fn = pl.kernel(
        _make_pass2(npair),
        out_type=jax.ShapeDtypeStruct((NC, npair + 1, NP, PW), jnp.float32),
        mesh=plsc.VectorSubcoreMesh(core_axis_name="c", subcore_axis_name="s"),
        scratch_types=[
            pltpu.VMEM((K2,), jnp.int32),
            pltpu.VMEM((K2,), jnp.int32),
            pltpu.VMEM((K2,), jnp.int32),
            pltpu.VMEM((K2, PW), jnp.float32),
            pltpu.VMEM((K2, PW), jnp.float32),
            pltpu.VMEM((K2, PW), jnp.float32),
            pltpu.VMEM((K2, 16), jnp.float32),
            pltpu.VMEM((16,), jnp.float32),
            pltpu.VMEM((K2, PW), jnp.float32),
            pltpu.VMEM_SHARED((AROWS, PW), jnp.float32),
            pltpu.SemaphoreType.DMA,
        ],
    )
    return fn(*xls, src, dst, alpha, gmax16)


def _gat_cat(p_ref, cb_ref, heads):
    npair = max(heads // 2, 1)
    den = p_ref[0, npair] + p_ref[1, npair]
    parts = []
    for p in range(npair):
        nf = p_ref[0, p] + p_ref[1, p]
        parts.append(nf[:, 0:D] / (den[:, 2 * p:2 * p + 1] + 1e-16))
        if heads > 1:
            parts.append(nf[:, D:PW] / (den[:, 2 * p + 1:2 * p + 2] + 1e-16))
    o = parts[0] if heads == 1 else jnp.concatenate(parts, axis=1)
    return o + cb_ref[...]


def _ln_elu(o, g_ref, be_ref):
    mu = jnp.mean(o, axis=-1, keepdims=True)
    var = jnp.mean((o - mu) ** 2, axis=-1, keepdims=True)
    o = (o - mu) * lax.rsqrt(var + 1e-5) * g_ref[...] + be_ref[...]
    return jnp.where(o > 0, o, jnp.exp(o) - 1.0)


def _comb0_body(p_ref, cb_ref, g_ref, be_ref, o_ref):
    o_ref[...] = _ln_elu(_gat_cat(p_ref, cb_ref, 8), g_ref, be_ref)


def _comb1_body(p_ref, cb_ref, g_ref, be_ref, res_ref, o_ref):
    o_ref[...] = (_ln_elu(_gat_cat(p_ref, cb_ref, 8), g_ref, be_ref)
                  + res_ref[...])


def _combine01(pd, cb, g, be, res=None):
    blk = 256
    nslot = pd.shape[1]
    hh = (nslot - 1) * PW
    vec_spec = pl.BlockSpec((1, hh), lambda i: (0, 0))
    in_specs = [pl.BlockSpec((NC, nslot, blk, PW), lambda i: (0, 0, i, 0)),
                vec_spec, vec_spec, vec_spec]
    args = [pd, cb.reshape(1, -1), g.reshape(1, -1), be.reshape(1, -1)]
    body = _comb0_body
    if res is not None:
        in_specs.append(pl.BlockSpec((blk, hh), lambda i: (i, 0)))
        args.append(res)
        body = _comb1_body
    return pl.pallas_call(
        body,
        grid=(NP // blk,),
        in_specs=in_specs,
        out_specs=pl.BlockSpec((blk, hh), lambda i: (i, 0)),
        out_shape=jax.ShapeDtypeStruct((NP, hh), jnp.float32),
    )(*args)


def _comb2_body(p_ref, cb_ref, g_ref, be_ref, w1_ref, b1_ref, w2_ref,
                b2_ref, o_ref):
    o = _ln_elu(_gat_cat(p_ref, cb_ref, 1), g_ref, be_ref)
    hmid = jnp.maximum(
        jnp.dot(o, w1_ref[...], preferred_element_type=jnp.float32)
        + b1_ref[...], 0.0)
    res = jnp.dot(hmid, w2_ref[...], preferred_element_type=jnp.float32)
    o_ref[...] = res[:, 0] + b2_ref[0, 0]


def _combine2(pd, cb, g, be, w1, b1, w2, b2):
    blk = 256
    w2p = jnp.zeros((D // 2, 128), jnp.float32).at[:, 0:1].set(w2)
    vec_spec = pl.BlockSpec((1, D), lambda i: (0, 0))
    return pl.pallas_call(
        _comb2_body,
        grid=(NP // blk,),
        in_specs=[
            pl.BlockSpec((NC, 2, blk, PW), lambda i: (0, 0, i, 0)),
            vec_spec, vec_spec, vec_spec,
            pl.BlockSpec((D, D // 2), lambda i: (0, 0)),
            pl.BlockSpec((1, D // 2), lambda i: (0, 0)),
            pl.BlockSpec((D // 2, 128), lambda i: (0, 0)),
            pl.BlockSpec((1, 1), lambda i: (0, 0)),
        ],
        out_specs=pl.BlockSpec((blk,), lambda i: (i,)),
        out_shape=jax.ShapeDtypeStruct((NP,), jnp.float32),
    )(pd, cb.reshape(1, -1), g.reshape(1, -1), be.reshape(1, -1),
      w1, b1.reshape(1, -1), w2p, b2.reshape(1, 1))


def _gat_layer(xin, src, dst, wl, bl, wr, br, att, heads):
    xl3 = _pair_matmul(xin, wl, bl, heads)
    xr3 = _pair_matmul(xin, wr, br, heads)
    npair = xl3.shape[0]
    xls = [xl3[p] for p in range(npair)]
    xrs = [xr3[p] for p in range(npair)]
    alpha = _pass1(xls, xrs, att, src, dst, heads)
    gmax16 = jnp.max(_colmax(alpha), axis=0)
    return _pass2(xls, src, dst, alpha, gmax16)


def kernel(x, edge_index, Wl0, bl0, Wr0, br0, att0, cb0, g0, be0,
           Wl1, bl1, Wr1, br1, att1, cb1, g1, be1,
           Wl2, bl2, Wr2, br2, att2, cb2, g2, be2,
           Wc1, bc1, Wc2, bc2):
    xp = jnp.pad(x, ((0, NP - N_NODES), (0, 0)))
    loops = jnp.arange(N_NODES, dtype=jnp.int32)
    fill = jnp.full((E2P - E2,), DUMMY, jnp.int32)
    src = jnp.concatenate([edge_index[0], loops, fill])
    dst = jnp.concatenate([edge_index[1], loops, fill])

    pd0 = _gat_layer(xp, src, dst, Wl0, bl0, Wr0, br0, att0, 8)
    x1 = _combine01(pd0, cb0, g0, be0)
    pd1 = _gat_layer(x1, src, dst, Wl1, bl1, Wr1, br1, att1, 8)
    x2 = _combine01(pd1, cb1, g1, be1, res=x1)
    pd2 = _gat_layer(x2, src, dst, Wl2, bl2, Wr2, br2, att2, 1)
    out = _combine2(pd2, cb2, g2, be2, Wc1, bc1, Wc2, bc2)
    return out[:N_NODES]


# speedup vs baseline: 7.5322x; 1.2570x over previous
import jax
import jax.numpy as jnp
from jax import lax
from jax.experimental import pallas as pl
from jax.experimental.pallas import tpu as pltpu
from jax.experimental.pallas import tpu_sc as plsc

N_NODES = 10000
NP = 10240
DUMMY = 10000
E_RAW = 320000
E2 = E_RAW + N_NODES
E2P = 331776
NC, NS = 2, 16
NW = NC * NS
CPW = E2P // NW
K1 = 64
K2 = 128
D = 64
PW = 128
HNP = NP // 2
AROWS = 5632
TRASH = HNP


def _mm_body(x_ref, w_ref, b_ref, o_ref):
    o_ref[0] = jnp.dot(x_ref[...], w_ref[0],
                       preferred_element_type=jnp.float32) + b_ref[0]


def _pair_matmul(x, w, b, heads):
    indim = x.shape[1]
    blk = 512
    if heads == 1:
        w = jnp.pad(w, ((0, 0), (0, D)))
        b = jnp.pad(b, (0, D))
    npair = max(heads // 2, 1)
    w3 = w.reshape(indim, npair, PW).transpose(1, 0, 2)
    b3 = b.reshape(1, npair, PW).transpose(1, 0, 2)
    return pl.pallas_call(
        _mm_body,
        grid=(npair, NP // blk),
        in_specs=[
            pl.BlockSpec((blk, indim), lambda p, i: (i, 0)),
            pl.BlockSpec((1, indim, PW), lambda p, i: (p, 0, 0)),
            pl.BlockSpec((1, 1, PW), lambda p, i: (p, 0, 0)),
        ],
        out_specs=pl.BlockSpec((1, blk, PW), lambda p, i: (p, i, 0)),
        out_shape=jax.ShapeDtypeStruct((npair, NP, PW), jnp.float32),
    )(x, w3, b3)


def _colmax_body(a_ref, o_ref):
    @pl.when(pl.program_id(0) == 0)
    def _():
        o_ref[...] = jnp.full_like(o_ref, -jnp.inf)
    blk = a_ref.shape[0]
    m = jnp.max(a_ref[...].reshape(blk // 8, 8, 16), axis=0)
    o_ref[...] = jnp.maximum(o_ref[...], m)


def _colmax(alpha):
    blk = 4096
    return pl.pallas_call(
        _colmax_body,
        grid=(E2P // blk,),
        in_specs=[pl.BlockSpec((blk, 16), lambda i: (i, 0))],
        out_specs=pl.BlockSpec((8, 16), lambda i: (0, 0)),
        out_shape=jax.ShapeDtypeStruct((8, 16), jnp.float32),
    )(alpha)


def _make_pass1(heads, npair):
    def body(*refs):
        xl = refs[0:npair]
        xr = refs[npair:2 * npair]
        att_hbm, src_hbm, dst_hbm = refs[2 * npair:2 * npair + 3]
        alpha_hbm = refs[2 * npair + 3]
        srcv, dstv, gxl, gxr, attv, abuf, gsem, isem = refs[2 * npair + 4:]
        c = lax.axis_index("c")
        s = lax.axis_index("s")
        base0 = (c * NS + s) * CPW
        pltpu.sync_copy(att_hbm, attv)
        attvals = [[attv[h, pl.ds(q * 16, 16)] for q in range(4)]
                   for h in range(heads)]
        lanes = lax.iota(jnp.int32, 16)
        shuf = [lanes ^ bit for bit in (8, 4, 2, 1)]
        dnums = lax.GatherDimensionNumbers(
            offset_dims=(), collapsed_slice_dims=(0,), start_index_map=(0,))

        def hsum(v):
            for ix in shuf:
                v = v + lax.gather(
                    v, ix[:, None], dnums, (1,),
                    mode=lax.GatherScatterMode.PROMISE_IN_BOUNDS)
            return v

        def chunk(k, carry):
            base = base0 + k * K1
            icps = [pltpu.async_copy(src_hbm.at[pl.ds(base, K1)], srcv,
                                     isem),
                    pltpu.async_copy(dst_hbm.at[pl.ds(base, K1)], dstv,
                                     isem)]
            for cp in icps:
                cp.wait()
            cps = []
            for p in range(npair):
                cps.append(pltpu.async_copy(xl[p].at[srcv], gxl.at[p], gsem))
                cps.append(pltpu.async_copy(xr[p].at[dstv], gxr.at[p], gsem))
            for cp in cps:
                cp.wait()

            def edge(e, ecarry):
                av = jnp.zeros((16,), jnp.float32)
                for h in range(heads):
                    p, half = divmod(h, 2)
                    off = half * D
                    acc = jnp.zeros((16,), jnp.float32)
                    for q in range(4):
                        sl = pl.ds(off + q * 16, 16)
                        sv = gxl[p, e, sl] + gxr[p, e, sl]
                        lr = jnp.maximum(sv, 0.2 * sv)
                        acc = acc + lr * attvals[h][q]
                    av = jnp.where(lanes == h, hsum(acc), av)
                abuf[e] = av
                return ecarry

            lax.fori_loop(0, K1, edge, 0)
            pltpu.sync_copy(abuf, alpha_hbm.at[pl.ds(base, K1)])
            return carry

        lax.fori_loop(0, CPW // K1, chunk, 0)
    return body


def _pass1(xls, xrs, att, src, dst, heads):
    npair = len(xls)
    fn = pl.kernel(
        _make_pass1(heads, npair),
        out_type=jax.ShapeDtypeStruct((E2P, 16), jnp.float32),
        mesh=plsc.VectorSubcoreMesh(core_axis_name="c", subcore_axis_name="s"),
        scratch_types=[
            pltpu.VMEM((K1,), jnp.int32),
            pltpu.VMEM((K1,), jnp.int32),
            pltpu.VMEM((npair, K1, PW), jnp.float32),
            pltpu.VMEM((npair, K1, PW), jnp.float32),
            pltpu.VMEM((heads, D), jnp.float32),
            pltpu.VMEM((K1, 16), jnp.float32),
            pltpu.SemaphoreType.DMA,
            pltpu.SemaphoreType.DMA,
        ],
    )
    return fn(*xls, *xrs, att, src, dst)


def _make_pass2(npair):
    def body(*refs):
        xl = refs[0:npair]
        src_hbm, dst_hbm, alpha_hbm, gmax_hbm, outp_hbm = \
            refs[npair:npair + 5]
        (srcv, dstv, dstw, gbuf, sbuf, zbuf, exv, gmv, fbuf, acc, gsem,
         isem) = refs[npair + 5:]
        c = lax.axis_index("c")
        s = lax.axis_index("s")
        base0 = (c * NS + s) * CPW
        pltpu.sync_copy(gmax_hbm, gmv)
        gmvec = gmv[...]
        zeros16 = jnp.zeros((16,), jnp.float32)

        def zrow(i, carry):
            for qq in range(PW // 16):
                zbuf[i, pl.ds(qq * 16, 16)] = zeros16
                sbuf[i, pl.ds(qq * 16, 16)] = zeros16
            return carry

        lax.fori_loop(0, K2, zrow, 0)

        def zero_acc():
            def zcp(j, carry):
                pltpu.sync_copy(zbuf.at[pl.ds(0, 88)],
                                acc.at[pl.ds(s * 352 + j * 88, 88)])
                return carry
            lax.fori_loop(0, 352 // 88, zcp, 0)
            plsc.subcore_barrier()

        def flush_acc(slot, r):
            plsc.subcore_barrier()

            def fcp(j, carry):
                r0 = s * 320 + j * 80
                pltpu.sync_copy(acc.at[pl.ds(r0, 80)], fbuf.at[pl.ds(0, 80)])
                pltpu.sync_copy(fbuf.at[pl.ds(0, 80)],
                                outp_hbm.at[c, slot,
                                            pl.ds(r * HNP + r0, 80)])
                return carry

            lax.fori_loop(0, 320 // 80, fcp, 0)
            plsc.subcore_barrier()

        def remap_dst(lo):
            def remap(g, carry):
                d = dstv[pl.ds(g * 16, 16)] - lo
                oob = (d < 0) | (d >= HNP)
                dstw[pl.ds(g * 16, 16)] = jnp.where(oob, TRASH, d)
                return carry

            lax.fori_loop(0, K2 // 16, remap, 0)

        for r in range(2):
            lo = r * HNP

            zero_acc()

            def dchunk(k, carry):
                base = base0 + k * K2
                cps = [pltpu.async_copy(dst_hbm.at[pl.ds(base, K2)], dstv,
                                        isem),
                       pltpu.async_copy(alpha_hbm.at[pl.ds(base, K2)], exv,
                                        isem)]
                for cp in cps:
                    cp.wait()
                remap_dst(lo)

                def vexp(e, gc):
                    sbuf[e, pl.ds(0, 16)] = jnp.exp(exv[e] - gmvec)
                    return gc

                lax.fori_loop(0, K2, vexp, 0)
                pltpu.sync_copy(sbuf, acc.at[dstw], add=True)
                return carry

            lax.fori_loop(0, CPW // K2, dchunk, 0)
            flush_acc(npair, r)

            def zden(i, carry):
                sbuf[i, pl.ds(0, 16)] = zeros16
                return carry

            lax.fori_loop(0, K2, zden, 0)

            for p in range(npair):
                zero_acc()

                def chunk(k, carry):
                    base = base0 + k * K2
                    cps = [pltpu.async_copy(src_hbm.at[pl.ds(base, K2)],
                                            srcv, isem),
                           pltpu.async_copy(dst_hbm.at[pl.ds(base, K2)],
                                            dstv, isem),
                           pltpu.async_copy(alpha_hbm.at[pl.ds(base, K2)],
                                            exv, isem)]
                    for cp in cps:
                        cp.wait()
                    gcp = pltpu.async_copy(xl[p].at[srcv], gbuf, gsem)
                    remap_dst(lo)

                    def vexp(e, gc):
                        exv[e] = jnp.exp(exv[e] - gmvec)
                        return gc

                    lax.fori_loop(0, K2, vexp, 0)
                    gcp.wait()

                    def edge(e, ec):
                        ev = exv[e]
                        ex0 = ev[2 * p]
                        ex1 = ev[2 * p + 1]
                        for q in range(8):
                            exq = ex0 if q < 4 else ex1
                            sbuf[e, pl.ds(q * 16, 16)] = \
                                gbuf[e, pl.ds(q * 16, 16)] * exq
                        return ec

                    lax.fori_loop(0, K2, edge, 0)
                    pltpu.sync_copy(sbuf, acc.at[dstw], add=True)
                    return carry

                lax.fori_loop(0, CPW // K2, chunk, 0)
                flush_acc(p, r)
    return body


def _pass2(xls, src, dst, alpha, gmax16):
    npair = len(xls)
    fn = pl.kernel(
        _make_pass2(npair),
        out_type=jax.ShapeDtypeStruct((NC, npair + 1, NP, PW), jnp.float32),
        mesh=plsc.VectorSubcoreMesh(core_axis_name="c", subcore_axis_name="s"),
        scratch_types=[
            pltpu.VMEM((K2,), jnp.int32),
            pltpu.VMEM((K2,), jnp.int32),
            pltpu.VMEM((K2,), jnp.int32),
            pltpu.VMEM((K2, PW), jnp.float32),
            pltpu.VMEM((K2, PW), jnp.float32),
            pltpu.VMEM((K2, PW), jnp.float32),
            pltpu.VMEM((K2, 16), jnp.float32),
            pltpu.VMEM((16,), jnp.float32),
            pltpu.VMEM((K2, PW), jnp.float32),
            pltpu.VMEM_SHARED((AROWS, PW), jnp.float32),
            pltpu.SemaphoreType.DMA,
            pltpu.SemaphoreType.DMA,
        ],
    )
    return fn(*xls, src, dst, alpha, gmax16)


def _gat_cat(p_ref, cb_ref, heads):
    npair = max(heads // 2, 1)
    den = p_ref[0, npair] + p_ref[1, npair]
    parts = []
    for p in range(npair):
        nf = p_ref[0, p] + p_ref[1, p]
        parts.append(nf[:, 0:D] / (den[:, 2 * p:2 * p + 1] + 1e-16))
        if heads > 1:
            parts.append(nf[:, D:PW] / (den[:, 2 * p + 1:2 * p + 2] + 1e-16))
    o = parts[0] if heads == 1 else jnp.concatenate(parts, axis=1)
    return o + cb_ref[...]


def _ln_elu(o, g_ref, be_ref):
    mu = jnp.mean(o, axis=-1, keepdims=True)
    var = jnp.mean((o - mu) ** 2, axis=-1, keepdims=True)
    o = (o - mu) * lax.rsqrt(var + 1e-5) * g_ref[...] + be_ref[...]
    return jnp.where(o > 0, o, jnp.exp(o) - 1.0)


def _comb0_body(p_ref, cb_ref, g_ref, be_ref, o_ref):
    o_ref[...] = _ln_elu(_gat_cat(p_ref, cb_ref, 8), g_ref, be_ref)


def _comb1_body(p_ref, cb_ref, g_ref, be_ref, res_ref, o_ref):
    o_ref[...] = (_ln_elu(_gat_cat(p_ref, cb_ref, 8), g_ref, be_ref)
                  + res_ref[...])


def _combine01(pd, cb, g, be, res=None):
    blk = 256
    nslot = pd.shape[1]
    hh = (nslot - 1) * PW
    vec_spec = pl.BlockSpec((1, hh), lambda i: (0, 0))
    in_specs = [pl.BlockSpec((NC, nslot, blk, PW), lambda i: (0, 0, i, 0)),
                vec_spec, vec_spec, vec_spec]
    args = [pd, cb.reshape(1, -1), g.reshape(1, -1), be.reshape(1, -1)]
    body = _comb0_body
    if res is not None:
        in_specs.append(pl.BlockSpec((blk, hh), lambda i: (i, 0)))
        args.append(res)
        body = _comb1_body
    return pl.pallas_call(
        body,
        grid=(NP // blk,),
        in_specs=in_specs,
        out_specs=pl.BlockSpec((blk, hh), lambda i: (i, 0)),
        out_shape=jax.ShapeDtypeStruct((NP, hh), jnp.float32),
    )(*args)


def _comb2_body(p_ref, cb_ref, g_ref, be_ref, w1_ref, b1_ref, w2_ref,
                b2_ref, o_ref):
    o = _ln_elu(_gat_cat(p_ref, cb_ref, 1), g_ref, be_ref)
    hmid = jnp.maximum(
        jnp.dot(o, w1_ref[...], preferred_element_type=jnp.float32)
        + b1_ref[...], 0.0)
    res = jnp.dot(hmid, w2_ref[...], preferred_element_type=jnp.float32)
    o_ref[...] = res[:, 0] + b2_ref[0, 0]


def _combine2(pd, cb, g, be, w1, b1, w2, b2):
    blk = 256
    w2p = jnp.zeros((D // 2, 128), jnp.float32).at[:, 0:1].set(w2)
    vec_spec = pl.BlockSpec((1, D), lambda i: (0, 0))
    return pl.pallas_call(
        _comb2_body,
        grid=(NP // blk,),
        in_specs=[
            pl.BlockSpec((NC, 2, blk, PW), lambda i: (0, 0, i, 0)),
            vec_spec, vec_spec, vec_spec,
            pl.BlockSpec((D, D // 2), lambda i: (0, 0)),
            pl.BlockSpec((1, D // 2), lambda i: (0, 0)),
            pl.BlockSpec((D // 2, 128), lambda i: (0, 0)),
            pl.BlockSpec((1, 1), lambda i: (0, 0)),
        ],
        out_specs=pl.BlockSpec((blk,), lambda i: (i,)),
        out_shape=jax.ShapeDtypeStruct((NP,), jnp.float32),
    )(pd, cb.reshape(1, -1), g.reshape(1, -1), be.reshape(1, -1),
      w1, b1.reshape(1, -1), w2p, b2.reshape(1, 1))


def _gat_layer(xin, src, dst, wl, bl, wr, br, att, heads):
    xl3 = _pair_matmul(xin, wl, bl, heads)
    xr3 = _pair_matmul(xin, wr, br, heads)
    npair = xl3.shape[0]
    xls = [xl3[p] for p in range(npair)]
    xrs = [xr3[p] for p in range(npair)]
    alpha = _pass1(xls, xrs, att, src, dst, heads)
    gmax16 = jnp.max(_colmax(alpha), axis=0)
    return _pass2(xls, src, dst, alpha, gmax16)


def kernel(x, edge_index, Wl0, bl0, Wr0, br0, att0, cb0, g0, be0,
           Wl1, bl1, Wr1, br1, att1, cb1, g1, be1,
           Wl2, bl2, Wr2, br2, att2, cb2, g2, be2,
           Wc1, bc1, Wc2, bc2):
    xp = jnp.pad(x, ((0, NP - N_NODES), (0, 0)))
    loops = jnp.arange(N_NODES, dtype=jnp.int32)
    fill = jnp.full((E2P - E2,), DUMMY, jnp.int32)
    src = jnp.concatenate([edge_index[0], loops, fill])
    dst = jnp.concatenate([edge_index[1], loops, fill])

    pd0 = _gat_layer(xp, src, dst, Wl0, bl0, Wr0, br0, att0, 8)
    x1 = _combine01(pd0, cb0, g0, be0)
    pd1 = _gat_layer(x1, src, dst, Wl1, bl1, Wr1, br1, att1, 8)
    x2 = _combine01(pd1, cb1, g1, be1, res=x1)
    pd2 = _gat_layer(x2, src, dst, Wl2, bl2, Wr2, br2, att2, 1)
    out = _combine2(pd2, cb2, g2, be2, Wc1, bc1, Wc2, bc2)
    return out[:N_NODES]
